# baseline (speedup 1.0000x reference)
import jax
import jax.numpy as jnp
from jax.experimental import pallas as pl
import jax.experimental.pallas.tpu as pltpu

N = 8
D = 128
BB = 512


def _fusion_kernel(x_ref, wsd_ref, be_ref, w1t_ref, b1_ref, w2a_ref,
                   out_ref):
    be = be_ref[0, 0]
    dnt = (((1,), (1,)), ((), ()))

    wsd = wsd_ref[:]
    ac = [jax.lax.dot_general(wsd, x_ref[i], dnt,
                              preferred_element_type=jnp.float32)
          for i in range(N)]
    a8 = jnp.concatenate([ac[i][0:1] for i in range(N)], axis=0)
    c8 = jnp.concatenate([ac[i][1:2] for i in range(N)], axis=0)

    logits = jnp.repeat(a8, N, axis=0) + jnp.tile(c8, (N, 1)) + be
    ew = jax.nn.sigmoid(logits)
    p = jax.lax.broadcasted_iota(jnp.int32, (N * N, 1), 0)
    offdiag = (p // N) != (p % N)
    ew = jnp.where(offdiag, ew, 0.0)
    deg = 1.0 + jnp.sum(ew.reshape(N, N, BB), axis=0)
    rs = jax.lax.rsqrt(deg)
    A64 = ew * jnp.repeat(rs, N, axis=0) * jnp.tile(rs, (N, 1))
    A64 = jnp.where(offdiag, A64, jnp.tile(1.0 / deg, (N, 1)))

    w1t = w1t_ref[:].astype(jnp.bfloat16)
    w2a = w2a_ref[:].astype(jnp.bfloat16)

    A16 = A64.astype(jnp.bfloat16)

    def aggregate(m):
        outs = []
        for j in range(N):
            acc = A16[j:j + 1, :] * m[0]
            for i in range(1, N):
                q = i * N + j
                acc = acc + A16[q:q + 1, :] * m[i]
            outs.append(acc)
        return outs

    b16 = b1_ref[:].astype(jnp.bfloat16)
    m1 = [jax.lax.dot_general(w1t, x_ref[i].astype(jnp.bfloat16), dnt,
                              preferred_element_type=jnp.float32
                              ).astype(jnp.bfloat16) + b16
          for i in range(N)]
    h = [jax.nn.relu(v) for v in aggregate(m1)]

    s8 = jnp.sum(A64.reshape(N, N, BB), axis=0)
    agg2 = aggregate(h)
    dn = (((0,), (0,)), ((), ()))
    s16 = s8.astype(jnp.bfloat16)
    slabs = []
    for j in range(N):
        aug = jnp.concatenate([agg2[j], s16[j:j + 1, :]], axis=0)
        slabs.append(jax.lax.dot_general(
            aug, w2a, dn,
            preferred_element_type=jnp.float32))
    out_ref[...] = jnp.concatenate(slabs, axis=1).reshape(BB, N, D)


def kernel(features_list, W_edge, b_edge, W1, b1, W2, b2):
    B = features_list.shape[1]
    wsd = jnp.stack([W_edge[:D, 0], W_edge[D:, 0]], axis=0)
    be = b_edge.reshape(1, 1)
    w1t = W1.T
    w2a = jnp.concatenate([W2, b2.reshape(1, D)], axis=0)
    b1c = b1.reshape(D, 1)

    grid = (B // BB,)
    rep2 = lambda i: (0, 0)
    out = pl.pallas_call(
        _fusion_kernel,
        grid=grid,
        in_specs=[
            pl.BlockSpec((N, BB, D), lambda i: (0, i, 0)),
            pl.BlockSpec((2, D), rep2),
            pl.BlockSpec((1, 1), rep2),
            pl.BlockSpec((D, D), rep2),
            pl.BlockSpec((D, 1), rep2),
            pl.BlockSpec((D + 1, D), rep2),
        ],
        out_specs=pl.BlockSpec((BB, N, D), lambda i: (i, 0, 0)),
        out_shape=jax.ShapeDtypeStruct((B, N, D), jnp.float32),
        compiler_params=pltpu.CompilerParams(
            dimension_semantics=("parallel",),
        ),
    )(features_list, wsd, be, w1t, b1c, w2a)
    return out


# speedup vs baseline: 1.1801x; 1.1801x over previous
import jax
import jax.numpy as jnp
from jax.experimental import pallas as pl
import jax.experimental.pallas.tpu as pltpu

N = 8
D = 128
BB = 512


def _fusion_kernel(x_ref, wsd_ref, be_ref, w1_ref, b1_ref, w2_ref, b2_ref,
                   out_ref):
    be = be_ref[0, 0]
    dnt = (((1,), (1,)), ((), ()))

    wsd = wsd_ref[:]
    ac = [jax.lax.dot_general(wsd, x_ref[i], dnt,
                              preferred_element_type=jnp.float32)
          for i in range(N)]
    a8 = jnp.concatenate([ac[i][0:1] for i in range(N)], axis=0)
    c8 = jnp.concatenate([ac[i][1:2] for i in range(N)], axis=0)

    logits = jnp.repeat(a8, N, axis=0) + jnp.tile(c8, (N, 1)) + be
    ew = jax.nn.sigmoid(logits)
    p = jax.lax.broadcasted_iota(jnp.int32, (N * N, 1), 0)
    offdiag = (p // N) != (p % N)
    ew = jnp.where(offdiag, ew, 0.0)
    deg = 1.0 + jnp.sum(ew.reshape(N, N, BB), axis=0)
    rs = jax.lax.rsqrt(deg)
    A64 = ew * jnp.repeat(rs, N, axis=0) * jnp.tile(rs, (N, 1))
    A64 = jnp.where(offdiag, A64, jnp.tile(1.0 / deg, (N, 1)))

    w1 = w1_ref[:].astype(jnp.bfloat16)
    w2a = jnp.concatenate([w2_ref[:], b2_ref[:]],
                          axis=0).astype(jnp.bfloat16)

    A16 = A64.astype(jnp.bfloat16)

    def aggregate(m):
        outs = []
        for j in range(N):
            acc = A16[j:j + 1, :] * m[0]
            for i in range(1, N):
                q = i * N + j
                acc = acc + A16[q:q + 1, :] * m[i]
            outs.append(acc)
        return outs

    b16 = b1_ref[:].astype(jnp.bfloat16)
    dn1 = (((0,), (1,)), ((), ()))
    m1 = [jax.lax.dot_general(w1, x_ref[i].astype(jnp.bfloat16), dn1,
                              preferred_element_type=jnp.float32
                              ).astype(jnp.bfloat16) + b16
          for i in range(N)]
    h = [jax.nn.relu(v) for v in aggregate(m1)]

    s8 = jnp.sum(A64.reshape(N, N, BB), axis=0)
    agg2 = aggregate(h)
    dn = (((0,), (0,)), ((), ()))
    s16 = s8.astype(jnp.bfloat16)
    slabs = []
    for j in range(N):
        aug = jnp.concatenate([agg2[j], s16[j:j + 1, :]], axis=0)
        slabs.append(jax.lax.dot_general(
            aug, w2a, dn,
            preferred_element_type=jnp.float32))
    out_ref[...] = jnp.concatenate(slabs, axis=1).reshape(BB, N, D)


def kernel(features_list, W_edge, b_edge, W1, b1, W2, b2):
    B = features_list.shape[1]
    wsd = W_edge.reshape(2, D)
    be = b_edge.reshape(1, 1)
    b1c = b1.reshape(D, 1)
    b2r = b2.reshape(1, D)

    grid = (B // BB,)
    rep2 = lambda i: (0, 0)
    out = pl.pallas_call(
        _fusion_kernel,
        grid=grid,
        in_specs=[
            pl.BlockSpec((N, BB, D), lambda i: (0, i, 0)),
            pl.BlockSpec((2, D), rep2),
            pl.BlockSpec((1, 1), rep2),
            pl.BlockSpec((D, D), rep2),
            pl.BlockSpec((D, 1), rep2),
            pl.BlockSpec((D, D), rep2),
            pl.BlockSpec((1, D), rep2),
        ],
        out_specs=pl.BlockSpec((BB, N, D), lambda i: (i, 0, 0)),
        out_shape=jax.ShapeDtypeStruct((B, N, D), jnp.float32),
        compiler_params=pltpu.CompilerParams(
            dimension_semantics=("parallel",),
        ),
    )(features_list, wsd, be, W1, b1c, W2, b2r)
    return out


# speedup vs baseline: 1.2085x; 1.0240x over previous
import jax
import jax.numpy as jnp
from jax.experimental import pallas as pl
import jax.experimental.pallas.tpu as pltpu

N = 8
D = 128
BB = 1024


def _fusion_kernel(x_ref, wsd_ref, be_ref, w1_ref, b1_ref, w2_ref, b2_ref,
                   out_ref):
    be = be_ref[0, 0]
    dnt = (((1,), (1,)), ((), ()))

    wsd = wsd_ref[:]
    ac = [jax.lax.dot_general(wsd, x_ref[i], dnt,
                              preferred_element_type=jnp.float32)
          for i in range(N)]
    a8 = jnp.concatenate([ac[i][0:1] for i in range(N)], axis=0)
    c8 = jnp.concatenate([ac[i][1:2] for i in range(N)], axis=0)

    logits = jnp.repeat(a8, N, axis=0) + jnp.tile(c8, (N, 1)) + be
    ew = jax.nn.sigmoid(logits)
    p = jax.lax.broadcasted_iota(jnp.int32, (N * N, 1), 0)
    offdiag = (p // N) != (p % N)
    ew = jnp.where(offdiag, ew, 0.0)
    deg = 1.0 + jnp.sum(ew.reshape(N, N, BB), axis=0)
    rs = jax.lax.rsqrt(deg)
    A64 = ew * jnp.repeat(rs, N, axis=0) * jnp.tile(rs, (N, 1))
    A64 = jnp.where(offdiag, A64, jnp.tile(1.0 / deg, (N, 1)))

    w1 = w1_ref[:].astype(jnp.bfloat16)
    w2a = jnp.concatenate([w2_ref[:], b2_ref[:]],
                          axis=0).astype(jnp.bfloat16)

    A16 = A64.astype(jnp.bfloat16)

    def aggregate(m):
        outs = []
        for j in range(N):
            acc = A16[j:j + 1, :] * m[0]
            for i in range(1, N):
                q = i * N + j
                acc = acc + A16[q:q + 1, :] * m[i]
            outs.append(acc)
        return outs

    b16 = b1_ref[:].astype(jnp.bfloat16)
    dn1 = (((0,), (1,)), ((), ()))
    m1 = [jax.lax.dot_general(w1, x_ref[i].astype(jnp.bfloat16), dn1,
                              preferred_element_type=jnp.float32
                              ).astype(jnp.bfloat16) + b16
          for i in range(N)]
    h = [jax.nn.relu(v) for v in aggregate(m1)]

    s8 = jnp.sum(A64.reshape(N, N, BB), axis=0)
    agg2 = aggregate(h)
    dn = (((0,), (0,)), ((), ()))
    s16 = s8.astype(jnp.bfloat16)
    slabs = []
    for j in range(N):
        aug = jnp.concatenate([agg2[j], s16[j:j + 1, :]], axis=0)
        slabs.append(jax.lax.dot_general(
            aug, w2a, dn,
            preferred_element_type=jnp.float32))
    out_ref[...] = jnp.concatenate(slabs, axis=1).reshape(BB, N, D)


def kernel(features_list, W_edge, b_edge, W1, b1, W2, b2):
    B = features_list.shape[1]
    wsd = W_edge.reshape(2, D)
    be = b_edge.reshape(1, 1)
    b1c = b1.reshape(D, 1)
    b2r = b2.reshape(1, D)

    grid = (B // BB,)
    rep2 = lambda i: (0, 0)
    out = pl.pallas_call(
        _fusion_kernel,
        grid=grid,
        in_specs=[
            pl.BlockSpec((N, BB, D), lambda i: (0, i, 0)),
            pl.BlockSpec((2, D), rep2),
            pl.BlockSpec((1, 1), rep2),
            pl.BlockSpec((D, D), rep2),
            pl.BlockSpec((D, 1), rep2),
            pl.BlockSpec((D, D), rep2),
            pl.BlockSpec((1, D), rep2),
        ],
        out_specs=pl.BlockSpec((BB, N, D), lambda i: (i, 0, 0)),
        out_shape=jax.ShapeDtypeStruct((B, N, D), jnp.float32),
        compiler_params=pltpu.CompilerParams(
            dimension_semantics=("parallel",),
        ),
    )(features_list, wsd, be, W1, b1c, W2, b2r)
    return out
